# Initial kernel scaffold; baseline (speedup 1.0000x reference)
import jax
import jax.numpy as jnp
from jax.experimental import pallas as pl


def kernel(tweet, sentiment, tweet_table, sentiment_table):
    raise NotImplementedError("write your pallas kernel here")


# speedup vs baseline: 1.6350x; 1.6350x over previous
import functools

import jax
import jax.numpy as jnp
from jax import lax
from jax.experimental import pallas as pl
from jax.experimental.pallas import tpu as pltpu
from jax.experimental.pallas import tpu_sc as plsc

NC, NS, L = 2, 16, 16
NW = NC * NS


def _body(dim, tz, sz, bw,
          t_tab, s_tab, tweet_r, sent_r, out_r, ttab_v, stab_v, tidx, sidx,
          big, sem):
    wid = lax.axis_index("s") * NC + lax.axis_index("c")
    pltpu.sync_copy(t_tab, ttab_v)
    pltpu.sync_copy(s_tab, stab_v)
    pltpu.sync_copy(tweet_r.at[wid], tidx)
    pltpu.sync_copy(sent_r.at[wid], sidx)

    lanes = lax.iota(jnp.int32, L)

    def chunk(ch, _):
        base = pl.multiple_of(ch * L, L)
        r16 = lanes + base
        t16 = tidx[pl.ds(base, L)]
        s16 = sidx[pl.ds(base, L)]
        m = t16 == 0
        t16 = jnp.where(m, tz, t16)
        s16 = jnp.where(m, sz, s16)
        for c in range(dim):
            col = jnp.full((L,), c, jnp.int32)
            v = plsc.load_gather(ttab_v, [t16, col])
            plsc.store_scatter(big, [r16, col], v)
            colr = jnp.full((L,), dim + c, jnp.int32)
            w = plsc.load_gather(stab_v, [s16, col])
            plsc.store_scatter(big, [r16, colr], w)
        return _

    lax.fori_loop(0, bw // L, chunk, None)
    pltpu.sync_copy(big, out_r.at[wid])


def kernel(tweet, sentiment, tweet_table, sentiment_table):
    b = tweet.shape[0]
    dim = tweet_table.shape[1]
    tz = tweet_table.shape[0]
    sz = sentiment_table.shape[0]
    bw = b // NW

    zrow = jnp.zeros((1, dim), jnp.float32)
    t_aug = jnp.concatenate([tweet_table, zrow], axis=0)
    s_aug = jnp.concatenate([sentiment_table, zrow], axis=0)
    tweet_r = tweet.astype(jnp.int32).reshape(NW, bw)
    sent_r = sentiment.astype(jnp.int32).reshape(NW, bw)

    mesh = plsc.VectorSubcoreMesh(core_axis_name="c", subcore_axis_name="s")
    run = pl.kernel(
        functools.partial(_body, dim, tz, sz, bw),
        out_type=jax.ShapeDtypeStruct((NW, bw, 2 * dim), jnp.float32),
        mesh=mesh,
        scratch_types=[
            pltpu.VMEM((tz + 1, dim), jnp.float32),
            pltpu.VMEM((sz + 1, dim), jnp.float32),
            pltpu.VMEM((bw,), jnp.int32),
            pltpu.VMEM((bw,), jnp.int32),
            pltpu.VMEM((bw, 2 * dim), jnp.float32),
            pltpu.SemaphoreType.DMA,
        ],
        compiler_params=pltpu.CompilerParams(needs_layout_passes=False),
    )
    out = run(t_aug, s_aug, tweet_r, sent_r)
    return out.reshape(b, 2 * dim)


# speedup vs baseline: 1.8286x; 1.1184x over previous
import functools

import jax
import jax.numpy as jnp
from jax import lax
from jax.experimental import pallas as pl
from jax.experimental.pallas import tpu as pltpu
from jax.experimental.pallas import tpu_sc as plsc

NC, NS, L = 2, 16, 16
NW = NC * NS


def _body(dim, bw,
          t_tab, s_tab, tweet_r, sent_r, out_r, ttab_v, stab_v, tidx, sidx,
          big, sem):
    wid = lax.axis_index("s") * NC + lax.axis_index("c")
    cps = [
        pltpu.async_copy(t_tab, ttab_v, sem),
        pltpu.async_copy(s_tab, stab_v, sem),
        pltpu.async_copy(tweet_r.at[wid], tidx, sem),
        pltpu.async_copy(sent_r.at[wid], sidx, sem),
    ]
    for c in cps:
        c.wait()

    lanes = lax.iota(jnp.int32, L)

    def chunk(ch, _):
        base = pl.multiple_of(ch * L, L)
        t16 = tidx[pl.ds(base, L)]
        s16 = sidx[pl.ds(base, L)]
        m = t16 == 0
        trow = t16 * dim
        srow = s16 * dim
        obase = (lanes + base) * (2 * dim)
        zero = jnp.zeros((L,), jnp.float32)
        tv = [plsc.load_gather(ttab_v, [trow + c]) for c in range(dim)]
        tv = [jnp.where(m, zero, v) for v in tv]
        for c in range(dim):
            plsc.store_scatter(big, [obase + c], tv[c])
        sv = [plsc.load_gather(stab_v, [srow + c]) for c in range(dim)]
        sv = [jnp.where(m, zero, v) for v in sv]
        for c in range(dim):
            plsc.store_scatter(big, [obase + (dim + c)], sv[c])
        return _

    lax.fori_loop(0, bw // L, chunk, None)
    pltpu.sync_copy(big, out_r.at[wid])


def kernel(tweet, sentiment, tweet_table, sentiment_table):
    b = tweet.shape[0]
    dim = tweet_table.shape[1]
    bw = b // NW

    t_flat = tweet_table.reshape(-1)
    s_flat = sentiment_table.reshape(-1)
    tweet_r = tweet.astype(jnp.int32).reshape(NW, bw)
    sent_r = sentiment.astype(jnp.int32).reshape(NW, bw)

    mesh = plsc.VectorSubcoreMesh(core_axis_name="c", subcore_axis_name="s")
    run = pl.kernel(
        functools.partial(_body, dim, bw),
        out_type=jax.ShapeDtypeStruct((NW, bw * 2 * dim), jnp.float32),
        mesh=mesh,
        scratch_types=[
            pltpu.VMEM((t_flat.shape[0],), jnp.float32),
            pltpu.VMEM((s_flat.shape[0],), jnp.float32),
            pltpu.VMEM((bw,), jnp.int32),
            pltpu.VMEM((bw,), jnp.int32),
            pltpu.VMEM((bw * 2 * dim,), jnp.float32),
            pltpu.SemaphoreType.DMA,
        ],
        compiler_params=pltpu.CompilerParams(needs_layout_passes=False),
    )
    out = run(t_flat, s_flat, tweet_r, sent_r)
    return out.reshape(b, 2 * dim)
